# baseline (speedup 1.0000x reference)
import functools

import jax
import jax.numpy as jnp
from jax import lax
from jax.experimental import pallas as pl
from jax.experimental.pallas import tpu as pltpu
from jax.experimental.pallas import tpu_sc as plsc

VOCAB = 1000000
MAX_LEN = 200
DIM = 64
BATCH = 4096

NC = 2
NS = 16
NW = NC * NS
LANES = 16

BPW = BATCH // NW
LC = 2
NCHUNKS = MAX_LEN // LC
TR = DIM // 8
TCOLS = BATCH // 128
CG = DIM // LANES
TBP = 129


def _emb_body(x_hbm, table_hbm, pos_hbm, out_hbm,
              idx_all, pos_v, idx_t0, idx_t1, g0, g1,
              t00, t01, t10, t11,
              sg0, sg1, so0, so1):
    wid = lax.axis_index("s") * NC + lax.axis_index("c")
    b0 = wid * BPW

    pltpu.sync_copy(x_hbm.at[pl.ds(b0, BPW), :], idx_all)
    pltpu.sync_copy(pos_hbm, pos_v)

    idx_ts = (idx_t0, idx_t1)
    gbufs = (g0, g1)
    tbufs = ((t00, t01), (t10, t11))
    gsems = (sg0, sg1)
    osems = (so0, so1)

    iota = lax.iota(jnp.int32, LANES)
    zeros = jnp.zeros((LANES,), jnp.int32)
    r0s = [(16 * c + iota) >> 3 for c in range(CG)]
    r1s = [(16 * c + iota) & 7 for c in range(CG)]

    def build_idx_t(g, which):
        it = idx_ts[which]
        for dl in range(LC):
            col = zeros + (g * LC + dl)
            for gi in range(BPW // LANES):
                v = plsc.load_gather(idx_all, [iota + gi * LANES, col])
                it[dl, pl.ds(gi * LANES, LANES)] = v * 2

    def start_gather(which):
        it = idx_ts[which]
        gb = gbufs[which]
        for dl in range(LC):
            pltpu.async_copy(table_hbm.at[it.at[dl]], gb.at[dl], gsems[which])

    def wait_gather(which):
        it = idx_ts[which]
        gb = gbufs[which]
        for dl in range(LC):
            pltpu.make_async_copy(
                table_hbm.at[it.at[dl]], gb.at[dl], gsems[which]).wait()

    def wait_outs(which):
        for dl in range(LC):
            pltpu.make_async_copy(
                tbufs[which][dl].at[:, :, pl.ds(0, 128)],
                out_hbm.at[0, pl.ds(0, TR), 0],
                osems[which]).wait()

    def process_chunk(g, which):
        gb = gbufs[which]
        for dl in range(LC):
            l = g * LC + dl
            tb = tbufs[which][dl]
            pos_vecs = [pos_v[l, pl.ds(16 * c, LANES)] for c in range(CG)]

            @plsc.parallel_loop(0, BPW, step=1, unroll=8)
            def _rows(j):
                colj = zeros + j
                for c in range(CG):
                    v = gb[dl, j, pl.ds(16 * c, LANES)] + pos_vecs[c]
                    plsc.store_scatter(tb, [r0s[c], r1s[c], colj], v)
            pltpu.async_copy(tb.at[:, :, pl.ds(0, 128)],
                             out_hbm.at[l, pl.ds(0, TR), wid],
                             osems[which])

    build_idx_t(0, 0)
    start_gather(0)
    build_idx_t(1, 1)
    start_gather(1)

    def outer(og, _):
        for b in range(2):
            g = og * 2 + b
            wait_gather(b)

            @pl.when(g >= 2)
            def _w():
                wait_outs(b)

            process_chunk(g, b)

            @pl.when(og < NCHUNKS // 2 - 1)
            def _n():
                build_idx_t(g + 2, b)
                start_gather(b)
        return _
    lax.fori_loop(0, NCHUNKS // 2, outer, None)

    for b in range(2):
        wait_outs(b)


def _emb_call(x, table_lin, pos_emb):
    mesh = plsc.VectorSubcoreMesh(core_axis_name="c", subcore_axis_name="s")
    f = functools.partial(
        pl.kernel,
        mesh=mesh,
        out_type=jax.ShapeDtypeStruct((MAX_LEN, TR, TCOLS, 8, 128),
                                      jnp.float32),
        compiler_params=pltpu.CompilerParams(use_tc_tiling_on_sc=False,
                                             needs_layout_passes=False),
        scratch_types=[
            pltpu.VMEM((BPW, MAX_LEN), jnp.int32),
            pltpu.VMEM((MAX_LEN, DIM), jnp.float32),
            pltpu.VMEM((LC, BPW), jnp.int32),
            pltpu.VMEM((LC, BPW), jnp.int32),
            pltpu.VMEM((LC, BPW, DIM), jnp.float32),
            pltpu.VMEM((LC, BPW, DIM), jnp.float32),
            pltpu.VMEM((TR, 8, TBP), jnp.float32),
            pltpu.VMEM((TR, 8, TBP), jnp.float32),
            pltpu.VMEM((TR, 8, TBP), jnp.float32),
            pltpu.VMEM((TR, 8, TBP), jnp.float32),
            pltpu.SemaphoreType.DMA,
            pltpu.SemaphoreType.DMA,
            pltpu.SemaphoreType.DMA,
            pltpu.SemaphoreType.DMA,
        ],
    )(_emb_body)
    return f(x, table_lin, pos_emb)


TBK = 1024


def _tpose_body(in_ref, out_ref):
    out_ref[:, 0:DIM] = in_ref[...].T


def _table_relayout(class_embT):
    grid = (VOCAB + TBK - 1) // TBK
    return pl.pallas_call(
        _tpose_body,
        grid=(grid,),
        in_specs=[pl.BlockSpec((DIM, TBK), lambda i: (0, i))],
        out_specs=pl.BlockSpec((TBK, 2 * DIM), lambda i: (i, 0)),
        out_shape=jax.ShapeDtypeStruct((VOCAB, 2 * DIM), jnp.float32),
    )(class_embT)


def kernel(x, class_emb, pos_emb):
    xi = x.astype(jnp.int32)
    t128 = _table_relayout(class_emb.T)
    table2 = t128.reshape(2 * VOCAB, DIM)
    out5 = _emb_call(xi, table2, pos_emb)
    return out5.transpose(2, 4, 0, 1, 3).reshape(BATCH, MAX_LEN, DIM)


# speedup vs baseline: 1.9756x; 1.9756x over previous
import functools

import jax
import jax.numpy as jnp
from jax import lax
from jax.experimental import pallas as pl
from jax.experimental.pallas import tpu as pltpu
from jax.experimental.pallas import tpu_sc as plsc

VOCAB = 1000000
MAX_LEN = 200
DIM = 64
BATCH = 4096

NC = 2
NS = 16
NW = NC * NS
LANES = 16

BPW = BATCH // NW
LC = 2
NCHUNKS = MAX_LEN // LC
TR = DIM // 8
TCOLS = BATCH // 128
CG = DIM // LANES
TBP = 129


def _emb_body(x_hbm, table_hbm, pos_hbm, out_hbm,
              idx_all, pos_v, idx_t0, idx_t1, g0, g1,
              t00, t01, t10, t11,
              sg0, sg1, so0, so1):
    wid = lax.axis_index("s") * NC + lax.axis_index("c")
    b0 = wid * BPW

    pltpu.sync_copy(x_hbm.at[pl.ds(b0, BPW), :], idx_all)
    pltpu.sync_copy(pos_hbm, pos_v)

    idx_ts = (idx_t0, idx_t1)
    gbufs = (g0, g1)
    tbufs = ((t00, t01), (t10, t11))
    gsems = (sg0, sg1)
    osems = (so0, so1)

    iota = lax.iota(jnp.int32, LANES)
    zeros = jnp.zeros((LANES,), jnp.int32)
    r0s = [(16 * c + iota) >> 3 for c in range(CG)]
    r1s = [(16 * c + iota) & 7 for c in range(CG)]

    def build_idx_t(g, which):
        it = idx_ts[which]
        for dl in range(LC):
            col = zeros + (g * LC + dl)
            for gi in range(BPW // LANES):
                v = plsc.load_gather(idx_all, [iota + gi * LANES, col])
                it[dl, pl.ds(gi * LANES, LANES)] = v * 2

    def start_gather(which):
        it = idx_ts[which]
        gb = gbufs[which]
        for dl in range(LC):
            pltpu.async_copy(table_hbm.at[it.at[dl]], gb.at[dl], gsems[which])

    def wait_gather(which):
        it = idx_ts[which]
        gb = gbufs[which]
        for dl in range(LC):
            pltpu.make_async_copy(
                table_hbm.at[it.at[dl]], gb.at[dl], gsems[which]).wait()

    def wait_outs(which):
        for dl in range(LC):
            pltpu.make_async_copy(
                tbufs[which][dl].at[:, :, pl.ds(0, 128)],
                out_hbm.at[0, pl.ds(0, TR), 0],
                osems[which]).wait()

    def process_chunk(g, which):
        gb = gbufs[which]
        for dl in range(LC):
            l = g * LC + dl
            tb = tbufs[which][dl]
            pos_vecs = [pos_v[l, pl.ds(16 * c, LANES)] for c in range(CG)]

            @plsc.parallel_loop(0, BPW, step=1, unroll=8)
            def _rows(j):
                colj = zeros + j
                for c in range(CG):
                    v = gb[dl, j, pl.ds(16 * c, LANES)] + pos_vecs[c]
                    plsc.store_scatter(tb, [r0s[c], r1s[c], colj], v)
            pltpu.async_copy(tb.at[:, :, pl.ds(0, 128)],
                             out_hbm.at[l, pl.ds(0, TR), wid],
                             osems[which])

    build_idx_t(0, 0)
    start_gather(0)
    build_idx_t(1, 1)
    start_gather(1)

    def outer(og, _):
        for b in range(2):
            g = og * 2 + b
            wait_gather(b)

            @pl.when(g >= 2)
            def _w():
                wait_outs(b)

            process_chunk(g, b)

            @pl.when(og < NCHUNKS // 2 - 1)
            def _n():
                build_idx_t(g + 2, b)
                start_gather(b)
        return _
    lax.fori_loop(0, NCHUNKS // 2, outer, None)

    for b in range(2):
        wait_outs(b)


def _emb_call(x, table_lin, pos_emb):
    mesh = plsc.VectorSubcoreMesh(core_axis_name="c", subcore_axis_name="s")
    f = functools.partial(
        pl.kernel,
        mesh=mesh,
        out_type=jax.ShapeDtypeStruct((MAX_LEN, TR, TCOLS, 8, 128),
                                      jnp.float32),
        compiler_params=pltpu.CompilerParams(use_tc_tiling_on_sc=False,
                                             needs_layout_passes=False),
        scratch_types=[
            pltpu.VMEM((BPW, MAX_LEN), jnp.int32),
            pltpu.VMEM((MAX_LEN, DIM), jnp.float32),
            pltpu.VMEM((LC, BPW), jnp.int32),
            pltpu.VMEM((LC, BPW), jnp.int32),
            pltpu.VMEM((LC, BPW, DIM), jnp.float32),
            pltpu.VMEM((LC, BPW, DIM), jnp.float32),
            pltpu.VMEM((TR, 8, TBP), jnp.float32),
            pltpu.VMEM((TR, 8, TBP), jnp.float32),
            pltpu.VMEM((TR, 8, TBP), jnp.float32),
            pltpu.VMEM((TR, 8, TBP), jnp.float32),
            pltpu.SemaphoreType.DMA,
            pltpu.SemaphoreType.DMA,
            pltpu.SemaphoreType.DMA,
            pltpu.SemaphoreType.DMA,
        ],
    )(_emb_body)
    return f(x, table_lin, pos_emb)


TBK = 8192


def _tpose_body(in_ref, out_ref):
    out_ref[:, 0:DIM] = in_ref[...].T


def _table_relayout(class_embT):
    grid = (VOCAB + TBK - 1) // TBK
    return pl.pallas_call(
        _tpose_body,
        grid=(grid,),
        in_specs=[pl.BlockSpec((DIM, TBK), lambda i: (0, i))],
        out_specs=pl.BlockSpec((TBK, 2 * DIM), lambda i: (i, 0)),
        out_shape=jax.ShapeDtypeStruct((VOCAB, 2 * DIM), jnp.float32),
    )(class_embT)


def kernel(x, class_emb, pos_emb):
    xi = x.astype(jnp.int32)
    t128 = _table_relayout(class_emb.T)
    table2 = t128.reshape(2 * VOCAB, DIM)
    out5 = _emb_call(xi, table2, pos_emb)
    return out5.transpose(2, 4, 0, 1, 3).reshape(BATCH, MAX_LEN, DIM)


# speedup vs baseline: 2.0694x; 1.0475x over previous
import functools

import jax
import jax.numpy as jnp
from jax import lax
from jax.experimental import pallas as pl
from jax.experimental.pallas import tpu as pltpu
from jax.experimental.pallas import tpu_sc as plsc

VOCAB = 1000000
MAX_LEN = 200
DIM = 64
BATCH = 4096

NC = 2
NS = 16
NW = NC * NS
LANES = 16

BPW = BATCH // NW
LC = 2
NCHUNKS = MAX_LEN // LC
TR = DIM // 8
TCOLS = BATCH // 128
CG = DIM // LANES
TBP = 129


def _emb_body(x_hbm, table_hbm, pos_hbm, out_hbm,
              idx_all, pos_v, idx_t0, idx_t1, g0, g1,
              t00, t01, t10, t11,
              sg0, sg1, so0, so1):
    wid = lax.axis_index("s") * NC + lax.axis_index("c")
    b0 = wid * BPW

    pltpu.sync_copy(x_hbm.at[pl.ds(b0, BPW), :], idx_all)
    pltpu.sync_copy(pos_hbm, pos_v)

    idx_ts = (idx_t0, idx_t1)
    gbufs = (g0, g1)
    tbufs = ((t00, t01), (t10, t11))
    gsems = (sg0, sg1)
    osems = (so0, so1)

    iota = lax.iota(jnp.int32, LANES)
    zeros = jnp.zeros((LANES,), jnp.int32)
    r0s = [(16 * c + iota) >> 3 for c in range(CG)]
    r1s = [(16 * c + iota) & 7 for c in range(CG)]

    def build_idx_t(g, which):
        it = idx_ts[which]
        for dl in range(LC):
            col = zeros + (g * LC + dl)
            for gi in range(BPW // LANES):
                v = plsc.load_gather(idx_all, [iota + gi * LANES, col])
                it[dl, pl.ds(gi * LANES, LANES)] = v * 2

    def start_gather(which):
        it = idx_ts[which]
        gb = gbufs[which]
        for dl in range(LC):
            pltpu.async_copy(table_hbm.at[it.at[dl]], gb.at[dl], gsems[which])

    def wait_gather(which):
        it = idx_ts[which]
        gb = gbufs[which]
        for dl in range(LC):
            pltpu.make_async_copy(
                table_hbm.at[it.at[dl]], gb.at[dl], gsems[which]).wait()

    def wait_outs(which):
        for dl in range(LC):
            pltpu.make_async_copy(
                tbufs[which][dl].at[:, :, pl.ds(0, 128)],
                out_hbm.at[0, pl.ds(0, TR), 0],
                osems[which]).wait()

    def process_chunk(g, which):
        gb = gbufs[which]
        for dl in range(LC):
            l = g * LC + dl
            tb = tbufs[which][dl]
            pos_vecs = [pos_v[l, pl.ds(16 * c, LANES)] for c in range(CG)]

            @plsc.parallel_loop(0, BPW, step=1, unroll=8)
            def _rows(j):
                colj = zeros + j
                for c in range(CG):
                    v = gb[dl, j, pl.ds(16 * c, LANES)] + pos_vecs[c]
                    plsc.store_scatter(tb, [r0s[c], r1s[c], colj], v)
            pltpu.async_copy(tb.at[:, :, pl.ds(0, 128)],
                             out_hbm.at[l, pl.ds(0, TR), wid],
                             osems[which])

    build_idx_t(0, 0)
    start_gather(0)
    build_idx_t(1, 1)
    start_gather(1)

    def outer(og, _):
        for b in range(2):
            g = og * 2 + b
            wait_gather(b)

            @pl.when(g >= 2)
            def _w():
                wait_outs(b)

            process_chunk(g, b)

            @pl.when(og < NCHUNKS // 2 - 1)
            def _n():
                build_idx_t(g + 2, b)
                start_gather(b)
        return _
    lax.fori_loop(0, NCHUNKS // 2, outer, None)

    for b in range(2):
        wait_outs(b)


def _emb_call(x, table_lin, pos_emb):
    mesh = plsc.VectorSubcoreMesh(core_axis_name="c", subcore_axis_name="s")
    f = functools.partial(
        pl.kernel,
        mesh=mesh,
        out_type=jax.ShapeDtypeStruct((MAX_LEN, TR, TCOLS, 8, 128),
                                      jnp.float32),
        compiler_params=pltpu.CompilerParams(use_tc_tiling_on_sc=False,
                                             needs_layout_passes=False),
        scratch_types=[
            pltpu.VMEM((BPW, MAX_LEN), jnp.int32),
            pltpu.VMEM((MAX_LEN, DIM), jnp.float32),
            pltpu.VMEM((LC, BPW), jnp.int32),
            pltpu.VMEM((LC, BPW), jnp.int32),
            pltpu.VMEM((LC, BPW, DIM), jnp.float32),
            pltpu.VMEM((LC, BPW, DIM), jnp.float32),
            pltpu.VMEM((TR, 8, TBP), jnp.float32),
            pltpu.VMEM((TR, 8, TBP), jnp.float32),
            pltpu.VMEM((TR, 8, TBP), jnp.float32),
            pltpu.VMEM((TR, 8, TBP), jnp.float32),
            pltpu.SemaphoreType.DMA,
            pltpu.SemaphoreType.DMA,
            pltpu.SemaphoreType.DMA,
            pltpu.SemaphoreType.DMA,
        ],
    )(_emb_body)
    return f(x, table_lin, pos_emb)


TBK = 16384


def _tpose_body(in_ref, out_ref):
    out_ref[:, 0:DIM] = in_ref[...].T


def _table_relayout(class_embT):
    grid = (VOCAB + TBK - 1) // TBK
    return pl.pallas_call(
        _tpose_body,
        grid=(grid,),
        in_specs=[pl.BlockSpec((DIM, TBK), lambda i: (0, i))],
        out_specs=pl.BlockSpec((TBK, 2 * DIM), lambda i: (i, 0)),
        out_shape=jax.ShapeDtypeStruct((VOCAB, 2 * DIM), jnp.float32),
    )(class_embT)


def kernel(x, class_emb, pos_emb):
    xi = x.astype(jnp.int32)
    t128 = _table_relayout(class_emb.T)
    table2 = t128.reshape(2 * VOCAB, DIM)
    out5 = _emb_call(xi, table2, pos_emb)
    return out5.transpose(2, 4, 0, 1, 3).reshape(BATCH, MAX_LEN, DIM)


# speedup vs baseline: 2.1012x; 1.0154x over previous
import functools

import jax
import jax.numpy as jnp
from jax import lax
from jax.experimental import pallas as pl
from jax.experimental.pallas import tpu as pltpu
from jax.experimental.pallas import tpu_sc as plsc

VOCAB = 1000000
MAX_LEN = 200
DIM = 64
BATCH = 4096

NC = 2
NS = 16
NW = NC * NS
LANES = 16

BPW = BATCH // NW
LC = 2
NCHUNKS = MAX_LEN // LC
TR = DIM // 8
TCOLS = BATCH // 128
CG = DIM // LANES
TBP = 129


def _emb_body(x_hbm, table_hbm, pos_hbm, out_hbm,
              idx_all, pos_v, idx_t0, idx_t1, g0, g1,
              t00, t01, t10, t11,
              sg0, sg1, so0, so1):
    wid = lax.axis_index("s") * NC + lax.axis_index("c")
    b0 = wid * BPW

    pltpu.sync_copy(x_hbm.at[pl.ds(b0, BPW), :], idx_all)
    pltpu.sync_copy(pos_hbm, pos_v)

    idx_ts = (idx_t0, idx_t1)
    gbufs = (g0, g1)
    tbufs = ((t00, t01), (t10, t11))
    gsems = (sg0, sg1)
    osems = (so0, so1)

    iota = lax.iota(jnp.int32, LANES)
    zeros = jnp.zeros((LANES,), jnp.int32)
    r0s = [(16 * c + iota) >> 3 for c in range(CG)]
    r1s = [(16 * c + iota) & 7 for c in range(CG)]

    def build_idx_t(g, which):
        it = idx_ts[which]
        for dl in range(LC):
            col = zeros + (g * LC + dl)
            for gi in range(BPW // LANES):
                v = plsc.load_gather(idx_all, [iota + gi * LANES, col])
                it[dl, pl.ds(gi * LANES, LANES)] = v * 2

    def start_gather(which):
        it = idx_ts[which]
        gb = gbufs[which]
        for dl in range(LC):
            pltpu.async_copy(table_hbm.at[it.at[dl]], gb.at[dl], gsems[which])

    def wait_gather(which):
        it = idx_ts[which]
        gb = gbufs[which]
        for dl in range(LC):
            pltpu.make_async_copy(
                table_hbm.at[it.at[dl]], gb.at[dl], gsems[which]).wait()

    def wait_outs(which):
        for dl in range(LC):
            pltpu.make_async_copy(
                tbufs[which][dl].at[:, :, pl.ds(0, 128)],
                out_hbm.at[0, pl.ds(0, TR), 0],
                osems[which]).wait()

    def process_chunk(g, which):
        gb = gbufs[which]
        for dl in range(LC):
            l = g * LC + dl
            tb = tbufs[which][dl]
            pos_vecs = [pos_v[l, pl.ds(16 * c, LANES)] for c in range(CG)]

            @plsc.parallel_loop(0, BPW, step=1, unroll=8)
            def _rows(j):
                colj = zeros + j
                for c in range(CG):
                    v = gb[dl, j, pl.ds(16 * c, LANES)] + pos_vecs[c]
                    plsc.store_scatter(tb, [r0s[c], r1s[c], colj], v)
            pltpu.async_copy(tb.at[:, :, pl.ds(0, 128)],
                             out_hbm.at[l, pl.ds(0, TR), wid],
                             osems[which])

    build_idx_t(0, 0)
    start_gather(0)
    build_idx_t(1, 1)
    start_gather(1)

    def outer(og, _):
        for b in range(2):
            g = og * 2 + b
            wait_gather(b)

            @pl.when(g >= 2)
            def _w():
                wait_outs(b)

            process_chunk(g, b)

            @pl.when(og < NCHUNKS // 2 - 1)
            def _n():
                build_idx_t(g + 2, b)
                start_gather(b)
        return _
    lax.fori_loop(0, NCHUNKS // 2, outer, None)

    for b in range(2):
        wait_outs(b)


def _emb_call(x, table_lin, pos_emb):
    mesh = plsc.VectorSubcoreMesh(core_axis_name="c", subcore_axis_name="s")
    f = functools.partial(
        pl.kernel,
        mesh=mesh,
        out_type=jax.ShapeDtypeStruct((MAX_LEN, TR, TCOLS, 8, 128),
                                      jnp.float32),
        compiler_params=pltpu.CompilerParams(use_tc_tiling_on_sc=False,
                                             needs_layout_passes=False),
        scratch_types=[
            pltpu.VMEM((BPW, MAX_LEN), jnp.int32),
            pltpu.VMEM((MAX_LEN, DIM), jnp.float32),
            pltpu.VMEM((LC, BPW), jnp.int32),
            pltpu.VMEM((LC, BPW), jnp.int32),
            pltpu.VMEM((LC, BPW, DIM), jnp.float32),
            pltpu.VMEM((LC, BPW, DIM), jnp.float32),
            pltpu.VMEM((TR, 8, TBP), jnp.float32),
            pltpu.VMEM((TR, 8, TBP), jnp.float32),
            pltpu.VMEM((TR, 8, TBP), jnp.float32),
            pltpu.VMEM((TR, 8, TBP), jnp.float32),
            pltpu.SemaphoreType.DMA,
            pltpu.SemaphoreType.DMA,
            pltpu.SemaphoreType.DMA,
            pltpu.SemaphoreType.DMA,
        ],
    )(_emb_body)
    return f(x, table_lin, pos_emb)


TBK = 32768


def _tpose_body(in_ref, out_ref):
    out_ref[:, 0:DIM] = in_ref[...].T


def _table_relayout(class_embT):
    grid = (VOCAB + TBK - 1) // TBK
    return pl.pallas_call(
        _tpose_body,
        grid=(grid,),
        in_specs=[pl.BlockSpec((DIM, TBK), lambda i: (0, i))],
        out_specs=pl.BlockSpec((TBK, 2 * DIM), lambda i: (i, 0)),
        out_shape=jax.ShapeDtypeStruct((VOCAB, 2 * DIM), jnp.float32),
    )(class_embT)


def kernel(x, class_emb, pos_emb):
    xi = x.astype(jnp.int32)
    t128 = _table_relayout(class_emb.T)
    table2 = t128.reshape(2 * VOCAB, DIM)
    out5 = _emb_call(xi, table2, pos_emb)
    return out5.transpose(2, 4, 0, 1, 3).reshape(BATCH, MAX_LEN, DIM)
